# baseline (speedup 1.0000x reference)
import functools

import jax
import jax.numpy as jnp
from jax import lax
from jax.experimental import pallas as pl
from jax.experimental.pallas import tpu as pltpu, tpu_sc as plsc

BATCH = 4096
SEQ = 50
EMBED_DIM = 64
N = BATCH * SEQ

NUM_CORES = 2
NUM_SUBCORES = 16
NUM_WORKERS = NUM_CORES * NUM_SUBCORES
ROWS_PER_WORKER = BATCH // NUM_WORKERS
CHUNK_B = 1
CHUNK = CHUNK_B * SEQ
NUM_CHUNKS = ROWS_PER_WORKER // CHUNK_B
NUM_PAIRS = NUM_CHUNKS // 2
LANES = 16
NUM_BINS = 12
GROUP_STARTS = (0, 16, 32, 34)


def _emb_sum_kernel(iw_hbm, is_hbm, words_hbm, scores_hbm, out_hbm,
                    idxw_v, idxs_v, stab,
                    wbuf0, wbuf1, obuf0, obuf1,
                    semw0, semw1, semo0, semo1):
    wid = lax.axis_index("s") * NUM_CORES + lax.axis_index("c")
    b_base = wid * ROWS_PER_WORKER
    wbuf = (wbuf0, wbuf1)
    obuf = (obuf0, obuf1)
    semw = (semw0, semw1)
    semo = (semo0, semo1)

    pltpu.sync_copy(iw_hbm.at[pl.ds(b_base, ROWS_PER_WORKER), :], idxw_v)
    pltpu.sync_copy(is_hbm.at[pl.ds(b_base, ROWS_PER_WORKER), :], idxs_v)
    pltpu.sync_copy(scores_hbm, stab)

    def gather_into(c, p):
        pltpu.async_copy(
            words_hbm.at[idxw_v.at[c]], wbuf[p], semw[p])

    gather_into(0, 0)
    gather_into(1, 1)

    def pair_body(i, carry):
        for p in range(2):
            c = i * 2 + p
            bb0 = c * CHUNK_B
            pltpu.make_async_copy(
                words_hbm.at[idxw_v.at[c]], wbuf[p], semw[p]).wait()

            @pl.when(i >= 1)
            def _wait_prev_scatter():
                pltpu.make_async_copy(
                    obuf[p], out_hbm.at[b_base + c], semo[p]).wait()

            for g in GROUP_STARTS:
                sidv = idxs_v[c, pl.ds(g, LANES)]
                for k in range(LANES):
                    sid = sidv[k]
                    for j in range(EMBED_DIM // LANES):
                        sl = pl.ds(j * LANES, LANES)
                        obuf[p][g + k, sl] = (
                            wbuf[p][g + k, sl] + stab[sid, sl])

            pltpu.async_copy(obuf[p], out_hbm.at[b_base + c], semo[p])

            @pl.when(i < NUM_PAIRS - 1)
            def _prefetch():
                gather_into(c + 2, p)
        return carry

    lax.fori_loop(0, NUM_PAIRS, pair_body, 0)

    for p in range(2):
        pltpu.make_async_copy(
            obuf[p], out_hbm.at[b_base + NUM_CHUNKS - 2 + p],
            semo[p]).wait()


@jax.jit
def kernel(input_ids, scores_ids, words_table, scores_table):
    iw = input_ids.astype(jnp.int32)
    isc = scores_ids.astype(jnp.int32)
    mesh = plsc.VectorSubcoreMesh(core_axis_name="c", subcore_axis_name="s")
    run = functools.partial(
        pl.kernel,
        mesh=mesh,
        compiler_params=pltpu.CompilerParams(use_tc_tiling_on_sc=False),
        out_type=jax.ShapeDtypeStruct((BATCH, SEQ, EMBED_DIM), jnp.float32),
        scratch_types=[
            pltpu.VMEM((ROWS_PER_WORKER, SEQ), jnp.int32),
            pltpu.VMEM((ROWS_PER_WORKER, SEQ), jnp.int32),
            pltpu.VMEM((NUM_BINS, EMBED_DIM), jnp.float32),
            pltpu.VMEM((SEQ, EMBED_DIM), jnp.float32),
            pltpu.VMEM((SEQ, EMBED_DIM), jnp.float32),
            pltpu.VMEM((SEQ, EMBED_DIM), jnp.float32),
            pltpu.VMEM((SEQ, EMBED_DIM), jnp.float32),
            pltpu.SemaphoreType.DMA,
            pltpu.SemaphoreType.DMA,
            pltpu.SemaphoreType.DMA,
            pltpu.SemaphoreType.DMA,
        ],
    )(_emb_sum_kernel)
    return run(iw, isc, words_table, scores_table)
